# baseline (speedup 1.0000x reference)
import functools
import math

import jax
import jax.numpy as jnp
from jax import lax
from jax.experimental import pallas as pl
from jax.experimental.pallas import tpu as pltpu
from jax.experimental.pallas import tpu_sc as plsc

NUM_MIX = 16
_HALF_LOG_2PI = 0.5 * math.log(2.0 * math.pi)


def _table_prep_tc(p_ref, s_ref, k_ref, si_ref):
    k_ref[...] = jnp.log(p_ref[...]) - jnp.log(s_ref[...]) - _HALF_LOG_2PI
    si_ref[...] = 1.0 / s_ref[...]


@functools.partial(jax.jit, static_argnums=(2, 3))
def _sc_logpdf(args, tables, n, lanes):
    z, x = args
    k16, mu16, si16 = tables
    info = plsc.get_sparse_core_info()
    nw = info.num_cores * info.num_subcores
    per_w = n // nw
    chunk = 8192
    n_chunks = per_w // chunk
    mesh = plsc.VectorSubcoreMesh(core_axis_name="c", subcore_axis_name="s")

    @functools.partial(
        pl.kernel,
        out_type=jax.ShapeDtypeStruct((n,), jnp.float32),
        mesh=mesh,
        compiler_params=pltpu.CompilerParams(needs_layout_passes=False),
        scratch_types=[
            pltpu.VMEM((NUM_MIX,), jnp.float32),
            pltpu.VMEM((NUM_MIX,), jnp.float32),
            pltpu.VMEM((NUM_MIX,), jnp.float32),
            pltpu.VMEM((chunk,), jnp.int32),
            pltpu.VMEM((chunk,), jnp.float32),
            pltpu.VMEM((chunk,), jnp.float32),
        ],
    )
    def sc_kernel(z_hbm, x_hbm, k_hbm, mu_hbm, si_hbm, out_hbm,
                  k_v, mu_v, si_v, z_v, x_v, o_v):
        wid = lax.axis_index("s") * info.num_cores + lax.axis_index("c")
        base0 = wid * per_w
        pltpu.sync_copy(k_hbm, k_v)
        pltpu.sync_copy(mu_hbm, mu_v)
        pltpu.sync_copy(si_hbm, si_v)

        def inner(j, _):
            zv = z_v[pl.ds(j * lanes, lanes)]
            xv = x_v[pl.ds(j * lanes, lanes)]
            kg = plsc.load_gather(k_v, [zv])
            mg = plsc.load_gather(mu_v, [zv])
            sg = plsc.load_gather(si_v, [zv])
            t = (xv - mg) * sg
            o_v[pl.ds(j * lanes, lanes)] = kg - 0.5 * t * t
            return 0

        for g in range(n_chunks):
            base = base0 + g * chunk
            pltpu.sync_copy(z_hbm.at[pl.ds(base, chunk)], z_v)
            pltpu.sync_copy(x_hbm.at[pl.ds(base, chunk)], x_v)
            lax.fori_loop(0, chunk // lanes, inner, 0)
            pltpu.sync_copy(o_v, out_hbm.at[pl.ds(base, chunk)])

    return sc_kernel(z, x, k16, mu16, si16)


def kernel(z, x, mixture_probs, means, stds):
    n = z.shape[0]

    p8 = jnp.ones((8, 128), jnp.float32).at[0, :NUM_MIX].set(mixture_probs)
    s8 = jnp.ones((8, 128), jnp.float32).at[0, :NUM_MIX].set(stds)
    k8, si8 = pl.pallas_call(
        _table_prep_tc,
        out_shape=(
            jax.ShapeDtypeStruct((8, 128), jnp.float32),
            jax.ShapeDtypeStruct((8, 128), jnp.float32),
        ),
    )(p8, s8)
    k16 = k8[0, :NUM_MIX]
    si16 = si8[0, :NUM_MIX]

    info = plsc.get_sparse_core_info()
    return _sc_logpdf((z, x), (k16, means.astype(jnp.float32), si16),
                      n, info.num_lanes)


# speedup vs baseline: 2.8305x; 2.8305x over previous
import functools
import math

import jax
import jax.numpy as jnp
from jax import lax
from jax.experimental import pallas as pl
from jax.experimental.pallas import tpu as pltpu
from jax.experimental.pallas import tpu_sc as plsc

NUM_MIX = 16
_HALF_LOG_2PI = 0.5 * math.log(2.0 * math.pi)


def _table_prep_tc(p_ref, s_ref, k_ref, si_ref):
    k_ref[...] = jnp.log(p_ref[...]) - jnp.log(s_ref[...]) - _HALF_LOG_2PI
    si_ref[...] = 1.0 / s_ref[...]


@functools.partial(jax.jit, static_argnums=(2, 3))
def _sc_logpdf(args, tables, n, lanes):
    z, x = args
    k16, mu16, si16 = tables
    info = plsc.get_sparse_core_info()
    nw = info.num_cores * info.num_subcores
    per_w = n // nw
    chunk = 16384
    n_chunks = per_w // chunk
    mesh = plsc.VectorSubcoreMesh(core_axis_name="c", subcore_axis_name="s")

    @functools.partial(
        pl.kernel,
        out_type=jax.ShapeDtypeStruct((n,), jnp.float32),
        mesh=mesh,
        compiler_params=pltpu.CompilerParams(needs_layout_passes=False),
        scratch_types=[
            pltpu.VMEM((NUM_MIX,), jnp.float32),
            pltpu.VMEM((NUM_MIX,), jnp.float32),
            pltpu.VMEM((NUM_MIX,), jnp.float32),
            [pltpu.VMEM((chunk,), jnp.int32) for _ in range(2)],
            [pltpu.VMEM((chunk,), jnp.float32) for _ in range(2)],
            [pltpu.VMEM((chunk,), jnp.float32) for _ in range(2)],
            [pltpu.SemaphoreType.DMA for _ in range(2)],
            [pltpu.SemaphoreType.DMA for _ in range(2)],
        ],
    )
    def sc_kernel(z_hbm, x_hbm, k_hbm, mu_hbm, si_hbm, out_hbm,
                  k_v, mu_v, si_v, z_v, x_v, o_v, in_sem, out_sem):
        wid = lax.axis_index("s") * info.num_cores + lax.axis_index("c")
        base0 = wid * per_w
        pltpu.sync_copy(k_hbm, k_v)
        pltpu.sync_copy(mu_hbm, mu_v)
        pltpu.sync_copy(si_hbm, si_v)

        def start_in(g):
            b = g % 2
            base = base0 + g * chunk
            dz = pltpu.async_copy(z_hbm.at[pl.ds(base, chunk)], z_v[b],
                                  in_sem[b])
            dx = pltpu.async_copy(x_hbm.at[pl.ds(base, chunk)], x_v[b],
                                  in_sem[b])
            return (dz, dx)

        in_flight = start_in(0)
        out_flight = [None, None]
        for g in range(n_chunks):
            b = g % 2
            for d in in_flight:
                d.wait()
            if g + 1 < n_chunks:
                in_flight = start_in(g + 1)
            if out_flight[b] is not None:
                out_flight[b].wait()

            @plsc.parallel_loop(0, chunk, lanes, unroll=8)
            def _(i):
                zv = z_v[b][pl.ds(i, lanes)]
                xv = x_v[b][pl.ds(i, lanes)]
                kg = plsc.load_gather(k_v, [zv])
                mg = plsc.load_gather(mu_v, [zv])
                sg = plsc.load_gather(si_v, [zv])
                t = (xv - mg) * sg
                o_v[b][pl.ds(i, lanes)] = kg - 0.5 * t * t

            out_flight[b] = pltpu.async_copy(
                o_v[b], out_hbm.at[pl.ds(base0 + g * chunk, chunk)],
                out_sem[b])
        for d in out_flight:
            if d is not None:
                d.wait()

    return sc_kernel(z, x, k16, mu16, si16)


def kernel(z, x, mixture_probs, means, stds):
    n = z.shape[0]

    p8 = jnp.ones((8, 128), jnp.float32).at[0, :NUM_MIX].set(mixture_probs)
    s8 = jnp.ones((8, 128), jnp.float32).at[0, :NUM_MIX].set(stds)
    k8, si8 = pl.pallas_call(
        _table_prep_tc,
        out_shape=(
            jax.ShapeDtypeStruct((8, 128), jnp.float32),
            jax.ShapeDtypeStruct((8, 128), jnp.float32),
        ),
    )(p8, s8)
    k16 = k8[0, :NUM_MIX]
    si16 = si8[0, :NUM_MIX]

    info = plsc.get_sparse_core_info()
    return _sc_logpdf((z, x), (k16, means.astype(jnp.float32), si16),
                      n, info.num_lanes)


# speedup vs baseline: 2.9999x; 1.0599x over previous
import functools
import math

import jax
import jax.numpy as jnp
from jax import lax
from jax.experimental import pallas as pl
from jax.experimental.pallas import tpu as pltpu
from jax.experimental.pallas import tpu_sc as plsc

NUM_MIX = 16
_HALF_LOG_2PI = 0.5 * math.log(2.0 * math.pi)
_LN2 = math.log(2.0)


def _vlog(v):
    bits = plsc.bitcast(v, jnp.int32)
    e = jnp.right_shift(bits, 23) - 127
    m_bits = jnp.bitwise_or(jnp.bitwise_and(bits, 0x007FFFFF), 0x3F800000)
    m = plsc.bitcast(m_bits, jnp.float32)
    y = (m - 1.0) / (m + 1.0)
    y2 = y * y
    ln_m = y * (2.0 + y2 * (2.0 / 3.0 + y2 * (2.0 / 5.0 + y2 * (
        2.0 / 7.0 + y2 * (2.0 / 9.0)))))
    return e.astype(jnp.float32) * _LN2 + ln_m


@functools.partial(jax.jit, static_argnums=(1, 2))
def _sc_logpdf(args, n, lanes):
    z, x, probs, mu, sd = args
    info = plsc.get_sparse_core_info()
    nw = info.num_cores * info.num_subcores
    per_w = n // nw
    chunk = 16384
    n_chunks = per_w // chunk
    mesh = plsc.VectorSubcoreMesh(core_axis_name="c", subcore_axis_name="s")

    @functools.partial(
        pl.kernel,
        out_type=jax.ShapeDtypeStruct((n,), jnp.float32),
        mesh=mesh,
        compiler_params=pltpu.CompilerParams(needs_layout_passes=False),
        scratch_types=[
            pltpu.VMEM((NUM_MIX,), jnp.float32),
            pltpu.VMEM((NUM_MIX,), jnp.float32),
            pltpu.VMEM((NUM_MIX,), jnp.float32),
            [pltpu.VMEM((chunk,), jnp.int32) for _ in range(2)],
            [pltpu.VMEM((chunk,), jnp.float32) for _ in range(2)],
            [pltpu.VMEM((chunk,), jnp.float32) for _ in range(2)],
            [pltpu.SemaphoreType.DMA for _ in range(2)],
            [pltpu.SemaphoreType.DMA for _ in range(2)],
        ],
    )
    def sc_kernel(z_hbm, x_hbm, p_hbm, mu_hbm, sd_hbm, out_hbm,
                  k_v, mu_v, si_v, z_v, x_v, o_v, in_sem, out_sem):
        wid = lax.axis_index("s") * info.num_cores + lax.axis_index("c")
        base0 = wid * per_w

        def start_in(g):
            b = g % 2
            base = base0 + g * chunk
            dz = pltpu.async_copy(z_hbm.at[pl.ds(base, chunk)], z_v[b],
                                  in_sem[b])
            dx = pltpu.async_copy(x_hbm.at[pl.ds(base, chunk)], x_v[b],
                                  in_sem[b])
            return (dz, dx)

        in_flight = start_in(0)

        pltpu.sync_copy(mu_hbm, mu_v)
        pltpu.sync_copy(sd_hbm, si_v)
        pltpu.sync_copy(p_hbm, k_v)
        sd_vec = si_v[...]
        ln_sd = _vlog(sd_vec)
        k_v[...] = _vlog(k_v[...]) - ln_sd - _HALF_LOG_2PI
        si_v[...] = 1.0 / sd_vec

        out_flight = [None, None]
        for g in range(n_chunks):
            b = g % 2
            for d in in_flight:
                d.wait()
            if g + 1 < n_chunks:
                in_flight = start_in(g + 1)
            if out_flight[b] is not None:
                out_flight[b].wait()

            @plsc.parallel_loop(0, chunk, lanes, unroll=8)
            def _(i):
                zv = z_v[b][pl.ds(i, lanes)]
                xv = x_v[b][pl.ds(i, lanes)]
                kg = plsc.load_gather(k_v, [zv])
                mg = plsc.load_gather(mu_v, [zv])
                sg = plsc.load_gather(si_v, [zv])
                t = (xv - mg) * sg
                o_v[b][pl.ds(i, lanes)] = kg - 0.5 * t * t

            out_flight[b] = pltpu.async_copy(
                o_v[b], out_hbm.at[pl.ds(base0 + g * chunk, chunk)],
                out_sem[b])
        for d in out_flight:
            if d is not None:
                d.wait()

    return sc_kernel(z, x, probs, mu, sd)


def kernel(z, x, mixture_probs, means, stds):
    n = z.shape[0]
    info = plsc.get_sparse_core_info()
    return _sc_logpdf(
        (z, x, mixture_probs.astype(jnp.float32),
         means.astype(jnp.float32), stds.astype(jnp.float32)),
        n, info.num_lanes)


# speedup vs baseline: 3.0812x; 1.0271x over previous
import functools
import math

import jax
import jax.numpy as jnp
from jax import lax
from jax.experimental import pallas as pl
from jax.experimental.pallas import tpu as pltpu
from jax.experimental.pallas import tpu_sc as plsc

NUM_MIX = 16
_HALF_LOG_2PI = 0.5 * math.log(2.0 * math.pi)
_LN2 = math.log(2.0)


def _vlog(v):
    bits = plsc.bitcast(v, jnp.int32)
    e = jnp.right_shift(bits, 23) - 127
    m_bits = jnp.bitwise_or(jnp.bitwise_and(bits, 0x007FFFFF), 0x3F800000)
    m = plsc.bitcast(m_bits, jnp.float32)
    y = (m - 1.0) / (m + 1.0)
    y2 = y * y
    ln_m = y * (2.0 + y2 * (2.0 / 3.0 + y2 * (2.0 / 5.0 + y2 * (
        2.0 / 7.0 + y2 * (2.0 / 9.0)))))
    return e.astype(jnp.float32) * _LN2 + ln_m


@functools.partial(jax.jit, static_argnums=(1, 2))
def _sc_logpdf(args, n, lanes):
    z, x, probs, mu, sd = args
    info = plsc.get_sparse_core_info()
    nw = info.num_cores * info.num_subcores
    per_w = n // nw
    chunk = 16384
    n_chunks = per_w // chunk
    mesh = plsc.VectorSubcoreMesh(core_axis_name="c", subcore_axis_name="s")

    @functools.partial(
        pl.kernel,
        out_type=jax.ShapeDtypeStruct((n,), jnp.float32),
        mesh=mesh,
        compiler_params=pltpu.CompilerParams(needs_layout_passes=False),
        scratch_types=[
            pltpu.VMEM((NUM_MIX,), jnp.float32),
            pltpu.VMEM((NUM_MIX,), jnp.float32),
            pltpu.VMEM((NUM_MIX,), jnp.float32),
            [pltpu.VMEM((chunk,), jnp.int32) for _ in range(2)],
            [pltpu.VMEM((chunk,), jnp.float32) for _ in range(2)],
            [pltpu.VMEM((chunk,), jnp.float32) for _ in range(2)],
            [pltpu.SemaphoreType.DMA for _ in range(2)],
            [pltpu.SemaphoreType.DMA for _ in range(2)],
        ],
    )
    def sc_kernel(z_hbm, x_hbm, p_hbm, mu_hbm, sd_hbm, out_hbm,
                  k_v, mu_v, si_v, z_v, x_v, o_v, in_sem, out_sem):
        wid = lax.axis_index("s") * info.num_cores + lax.axis_index("c")
        base0 = wid * per_w

        def start_in(g):
            b = g % 2
            base = base0 + g * chunk
            dz = pltpu.async_copy(z_hbm.at[pl.ds(base, chunk)], z_v[b],
                                  in_sem[b])
            dx = pltpu.async_copy(x_hbm.at[pl.ds(base, chunk)], x_v[b],
                                  in_sem[b])
            return (dz, dx)

        start_in(0)

        pltpu.sync_copy(mu_hbm, mu_v)
        pltpu.sync_copy(sd_hbm, si_v)
        pltpu.sync_copy(p_hbm, k_v)
        sd_vec = si_v[...]
        ln_sd = _vlog(sd_vec)
        k_v[...] = _vlog(k_v[...]) - ln_sd - _HALF_LOG_2PI
        si_v[...] = 1.0 / sd_vec

        start_in(1)

        @pl.loop(0, n_chunks, step=2)
        def _(g):
            for b in range(2):
                gg = g + b
                base = base0 + gg * chunk
                pltpu.make_async_copy(z_hbm.at[pl.ds(base, chunk)], z_v[b],
                                      in_sem[b]).wait()
                pltpu.make_async_copy(x_hbm.at[pl.ds(base, chunk)], x_v[b],
                                      in_sem[b]).wait()

                @pl.when(gg >= 2)
                def _():
                    pltpu.make_async_copy(
                        o_v[b], out_hbm.at[pl.ds(base0, chunk)],
                        out_sem[b]).wait()

                @plsc.parallel_loop(0, chunk, lanes, unroll=8)
                def _(i):
                    zv = z_v[b][pl.ds(i, lanes)]
                    xv = x_v[b][pl.ds(i, lanes)]
                    kg = plsc.load_gather(k_v, [zv])
                    mg = plsc.load_gather(mu_v, [zv])
                    sg = plsc.load_gather(si_v, [zv])
                    t = (xv - mg) * sg
                    o_v[b][pl.ds(i, lanes)] = kg - 0.5 * t * t

                pltpu.async_copy(
                    o_v[b], out_hbm.at[pl.ds(base, chunk)], out_sem[b])

                @pl.when(gg + 2 < n_chunks)
                def _():
                    nxt = base0 + (gg + 2) * chunk
                    pltpu.async_copy(z_hbm.at[pl.ds(nxt, chunk)], z_v[b],
                                     in_sem[b])
                    pltpu.async_copy(x_hbm.at[pl.ds(nxt, chunk)], x_v[b],
                                     in_sem[b])

        for b in range(2):
            pltpu.make_async_copy(
                o_v[b], out_hbm.at[pl.ds(base0, chunk)], out_sem[b]).wait()

    return sc_kernel(z, x, probs, mu, sd)


def kernel(z, x, mixture_probs, means, stds):
    n = z.shape[0]
    info = plsc.get_sparse_core_info()
    return _sc_logpdf(
        (z, x, mixture_probs.astype(jnp.float32),
         means.astype(jnp.float32), stds.astype(jnp.float32)),
        n, info.num_lanes)
